# baseline (speedup 1.0000x reference)
import functools

import jax
import jax.numpy as jnp
from jax import lax
from jax.experimental import pallas as pl
from jax.experimental.pallas import tpu as pltpu
from jax.experimental.pallas import tpu_sc as plsc

B = 16
S_LEN = 128
N_HASH = 64
M_BLOOM = 1024
W_WIN = 3
NBLK = 2 * W_WIN + 1

LANES = 16
NUM_CORES = 2
NUM_SUBCORES = 16
MH = M_BLOOM // 2
SBLKS = S_LEN // LANES


def _hist_body(mh_hbm, cnt_hbm, inp, cnt, sem):
    wid = lax.axis_index("s") * NUM_CORES + lax.axis_index("c")
    b = wid // 2
    m_base = (wid % 2) * MH

    in_copy = pltpu.make_async_copy(mh_hbm.at[b], inp, sem)
    in_copy.start()

    zeros = jnp.zeros((LANES,), jnp.float32)

    def zrow(r, _):
        for j in range(S_LEN // LANES):
            cnt[r, pl.ds(j * LANES, LANES)] = zeros
        return 0

    lax.fori_loop(0, MH, zrow, 0)
    in_copy.wait()

    iota = lax.iota(jnp.int32, LANES)
    ones = jnp.ones((LANES,), jnp.float32)

    def scat(i, _):
        n = i // SBLKS
        sb = i - n * SBLKS
        s_vec = sb * LANES + iota
        n_vec = jnp.full((LANES,), n, jnp.int32)
        h = plsc.load_gather(inp, [s_vec, n_vec])
        rel = (h & (M_BLOOM - 1)) - m_base
        mask = (rel >= 0) & (rel < MH)
        rel_safe = jnp.where(mask, rel, 0)
        plsc.addupdate_scatter(cnt, [rel_safe, s_vec], ones, mask=mask)
        return 0

    lax.fori_loop(0, N_HASH * SBLKS, scat, 0)

    pltpu.sync_copy(cnt, cnt_hbm.at[b, pl.ds(m_base, MH), :])


def _sc_histogram(minhashes):
    mesh = plsc.VectorSubcoreMesh(
        core_axis_name="c", subcore_axis_name="s",
        num_cores=NUM_CORES, num_subcores=NUM_SUBCORES,
    )
    run = pl.kernel(
        _hist_body,
        out_type=jax.ShapeDtypeStruct((B, M_BLOOM, S_LEN), jnp.float32),
        mesh=mesh,
        scratch_types=[
            pltpu.VMEM((S_LEN, N_HASH), jnp.int32),
            pltpu.VMEM((MH, S_LEN), jnp.float32),
            pltpu.SemaphoreType.DMA,
        ],
        compiler_params=pltpu.CompilerParams(
            use_tc_tiling_on_sc=False, needs_layout_passes=False
        ),
    )
    return run(minhashes)


MC = 256


def _emit_body(cin, cout):
    x = cin[0]
    for k in range(NBLK):
        d = W_WIN - k
        if d > 0:
            blk = jnp.concatenate(
                [jnp.zeros((MC, d), jnp.float32), x[:, : S_LEN - d]], axis=1
            )
        elif d == 0:
            blk = x
        else:
            e = -d
            blk = jnp.concatenate(
                [x[:, e:], jnp.zeros((MC, e), jnp.float32)], axis=1
            )
        cout[0, k] = blk


def _tc_emit(counts):
    out4 = pl.pallas_call(
        _emit_body,
        out_shape=jax.ShapeDtypeStruct((B, NBLK, M_BLOOM, S_LEN), jnp.float32),
        grid=(B, M_BLOOM // MC),
        in_specs=[pl.BlockSpec((1, MC, S_LEN), lambda i, j: (i, j, 0))],
        out_specs=pl.BlockSpec((1, NBLK, MC, S_LEN), lambda i, j: (i, 0, j, 0)),
    )(counts)
    return out4.reshape(B, NBLK * M_BLOOM, S_LEN)


@functools.partial(jax.jit, static_argnames=())
def kernel(minhashes):
    return _tc_emit(_sc_histogram(minhashes))
